# Initial kernel scaffold; baseline (speedup 1.0000x reference)
import jax
import jax.numpy as jnp
from jax.experimental import pallas as pl


def kernel(v, idx):
    raise NotImplementedError("write your pallas kernel here")


# speedup vs baseline: 135.6119x; 135.6119x over previous
import functools

import jax
import jax.numpy as jnp
from jax import lax
from jax.experimental import pallas as pl
from jax.experimental.pallas import tpu as pltpu
from jax.experimental.pallas import tpu_sc as plsc

_NC = 2
_NS = 16
_NW = _NC * _NS


def _gather_call(n_total, chunk):
    n_per_w = n_total // _NW
    n_chunks = n_per_w // chunk
    mesh = plsc.VectorSubcoreMesh(core_axis_name="c", subcore_axis_name="s")

    @functools.partial(
        pl.kernel,
        mesh=mesh,
        out_type=jax.ShapeDtypeStruct((n_total,), jnp.float32),
        scratch_types=[
            pltpu.VMEM((chunk,), jnp.int32),
            pltpu.VMEM((chunk,), jnp.float32),
            pltpu.SemaphoreType.DMA,
        ],
    )
    def k(v_hbm, idx_hbm, out_hbm, idx_v, val_v, sem):
        wid = lax.axis_index("s") * _NC + lax.axis_index("c")
        base = wid * n_per_w

        def body(i, carry):
            off = base + i * chunk
            pltpu.sync_copy(idx_hbm.at[pl.ds(off, chunk)], idx_v)
            pltpu.async_copy(v_hbm.at[idx_v], val_v, sem).wait()
            pltpu.sync_copy(val_v, out_hbm.at[pl.ds(off, chunk)])
            return carry

        lax.fori_loop(0, n_chunks, body, 0)

    return k


def kernel(v, idx):
    b, s = idx.shape
    n_total = b * s
    idx_flat = idx.reshape(n_total).astype(jnp.int32)
    out = _gather_call(n_total, 12800)(v, idx_flat)
    return out.reshape(b, s)


# speedup vs baseline: 206.2276x; 1.5207x over previous
import functools

import jax
import jax.numpy as jnp
from jax import lax
from jax.experimental import pallas as pl
from jax.experimental.pallas import tpu as pltpu
from jax.experimental.pallas import tpu_sc as plsc

_NC = 2
_NS = 16
_NW = _NC * _NS


def _gather_call(n_total, n_table, chunk):
    n_per_w = n_total // _NW
    n_chunks = n_per_w // chunk
    stage = 8000
    n_stage = n_table // stage
    mesh = plsc.VectorSubcoreMesh(core_axis_name="c", subcore_axis_name="s")

    @functools.partial(
        pl.kernel,
        mesh=mesh,
        out_type=jax.ShapeDtypeStruct((n_total,), jnp.float32),
        scratch_types=[
            pltpu.VMEM_SHARED((n_table,), jnp.float32),
            pltpu.VMEM((stage,), jnp.float32),
            pltpu.VMEM((chunk,), jnp.int32),
            pltpu.VMEM((chunk,), jnp.float32),
            pltpu.SemaphoreType.DMA,
        ],
    )
    def k(v_hbm, idx_hbm, out_hbm, tab_sp, stg_v, idx_v, val_v, sem):
        cid = lax.axis_index("c")
        sid = lax.axis_index("s")
        wid = sid * _NC + cid

        def stage_body(j, carry):
            c = sid + j * _NS

            @pl.when(c < n_stage)
            def _():
                off = c * stage
                pltpu.sync_copy(v_hbm.at[pl.ds(off, stage)], stg_v)
                pltpu.sync_copy(stg_v, tab_sp.at[pl.ds(off, stage)])

            return carry

        lax.fori_loop(0, (n_stage + _NS - 1) // _NS, stage_body, 0)

        plsc.subcore_barrier()

        base = wid * n_per_w

        def body(i, carry):
            off = base + i * chunk
            pltpu.sync_copy(idx_hbm.at[pl.ds(off, chunk)], idx_v)
            pltpu.async_copy(tab_sp.at[idx_v], val_v, sem).wait()
            pltpu.sync_copy(val_v, out_hbm.at[pl.ds(off, chunk)])
            return carry

        lax.fori_loop(0, n_chunks, body, 0)

    return k


def kernel(v, idx):
    b, s = idx.shape
    n_total = b * s
    idx_flat = idx.reshape(n_total).astype(jnp.int32)
    out = _gather_call(n_total, v.shape[0], 12800)(v, idx_flat)
    return out.reshape(b, s)


# speedup vs baseline: 288.1232x; 1.3971x over previous
import functools

import jax
import jax.numpy as jnp
from jax import lax
from jax.experimental import pallas as pl
from jax.experimental.pallas import tpu as pltpu
from jax.experimental.pallas import tpu_sc as plsc

_NC = 2
_NS = 16
_NW = _NC * _NS


def _gather_call(n_rows, n_cols, n_table, chunk_rows):
    rows_per_w = n_rows // _NW
    n_chunks = rows_per_w // chunk_rows
    stage = 8000
    n_stage = n_table // stage
    mesh = plsc.VectorSubcoreMesh(core_axis_name="c", subcore_axis_name="s")

    @functools.partial(
        pl.kernel,
        mesh=mesh,
        out_type=jax.ShapeDtypeStruct((n_rows, n_cols), jnp.float32),
        scratch_types=[
            pltpu.VMEM_SHARED((n_table,), jnp.float32),
            pltpu.VMEM((stage,), jnp.float32),
            pltpu.VMEM((chunk_rows, n_cols), jnp.int32),
            pltpu.VMEM((chunk_rows, n_cols), jnp.float32),
            pltpu.SemaphoreType.DMA,
            pltpu.SemaphoreType.DMA,
        ],
    )
    def k(v_hbm, idx2d_hbm, out2d_hbm, tab_sp, stg_v, idx_v, val_v, sem, sem2):
        n_chunk_total = n_rows // chunk_rows
        idx_hbm = idx2d_hbm.reshape(n_chunk_total, chunk_rows, n_cols)
        out_hbm = out2d_hbm.reshape(n_chunk_total, chunk_rows, n_cols)
        cid = lax.axis_index("c")
        sid = lax.axis_index("s")
        wid = sid * _NC + cid

        def stage_body(j, carry):
            c = sid + j * _NS

            @pl.when(c < n_stage)
            def _():
                off = c * stage
                pltpu.sync_copy(v_hbm.at[pl.ds(off, stage)], stg_v)
                pltpu.sync_copy(stg_v, tab_sp.at[pl.ds(off, stage)])

            return carry

        lax.fori_loop(0, (n_stage + _NS - 1) // _NS, stage_body, 0)

        plsc.subcore_barrier()

        base = wid * n_chunks
        rows_per_fire = 8
        seg0 = 128
        seg1 = n_cols - seg0

        def body(i, carry):
            c = base + i
            pltpu.sync_copy(idx_hbm.at[c], idx_v)

            def fire(g, carry2):
                descs = []
                for j in range(rows_per_fire):
                    r = g * rows_per_fire + j
                    descs.append(pltpu.async_copy(
                        tab_sp.at[idx_v.at[r, pl.ds(0, seg0)]],
                        val_v.at[r, pl.ds(0, seg0)],
                        sem,
                    ))
                    descs.append(pltpu.async_copy(
                        tab_sp.at[idx_v.at[r, pl.ds(seg0, seg1)]],
                        val_v.at[r, pl.ds(seg0, seg1)],
                        sem,
                    ))
                for d in descs:
                    d.wait()
                return carry2

            lax.fori_loop(0, chunk_rows // rows_per_fire, fire, 0)
            pltpu.sync_copy(val_v, out_hbm.at[c])
            return carry

        lax.fori_loop(0, n_chunks, body, 0)

    return k


def kernel(v, idx):
    b, s = idx.shape
    return _gather_call(b, s, v.shape[0], 64)(v, idx.astype(jnp.int32))


# speedup vs baseline: 353.5662x; 1.2271x over previous
import functools

import jax
import jax.numpy as jnp
from jax import lax
from jax.experimental import pallas as pl
from jax.experimental.pallas import tpu as pltpu
from jax.experimental.pallas import tpu_sc as plsc

_NC = 2
_NS = 16
_NW = _NC * _NS
_LANES = 128
_SUBL = 8


def _gather_call(n_rows, n_cols, n_table, block_cols):
    n_strips = n_rows // _SUBL
    blocks_per_strip = n_cols // block_cols
    n_blocks = n_strips * blocks_per_strip
    blocks_per_w = n_blocks // _NW
    segs = block_cols // _LANES
    stage = 8000
    n_stage = n_table // stage
    mesh = plsc.VectorSubcoreMesh(core_axis_name="c", subcore_axis_name="s")

    @functools.partial(
        pl.kernel,
        mesh=mesh,
        out_type=jax.ShapeDtypeStruct((n_rows, n_cols), jnp.float32),
        scratch_types=[
            pltpu.VMEM_SHARED((n_table,), jnp.float32),
            pltpu.VMEM((stage,), jnp.float32),
            pltpu.VMEM((_SUBL, block_cols), jnp.int32),
            pltpu.VMEM((_SUBL, block_cols), jnp.float32),
            pltpu.SemaphoreType.DMA,
        ],
    )
    def k(v_hbm, idx2d_hbm, out2d_hbm, tab_sp, stg_v, idx_v, val_v, sem):
        cid = lax.axis_index("c")
        sid = lax.axis_index("s")
        wid = sid * _NC + cid

        def stage_body(j, carry):
            c = sid + j * _NS

            @pl.when(c < n_stage)
            def _():
                off = c * stage
                pltpu.sync_copy(v_hbm.at[pl.ds(off, stage)], stg_v)
                pltpu.sync_copy(stg_v, tab_sp.at[pl.ds(off, stage)])

            return carry

        lax.fori_loop(0, (n_stage + _NS - 1) // _NS, stage_body, 0)

        plsc.subcore_barrier()

        idx_hbm = idx2d_hbm.reshape(n_strips, _SUBL, n_cols)
        out_hbm = out2d_hbm.reshape(n_strips, _SUBL, n_cols)

        base = wid * blocks_per_w

        def body(i, carry):
            q = base + i
            t = q // blocks_per_strip
            b = q % blocks_per_strip
            col0 = b * block_cols
            pltpu.sync_copy(
                idx_hbm.at[t, :, pl.ds(col0, block_cols)], idx_v
            )
            descs = []
            for r in range(_SUBL):
                for s in range(segs):
                    descs.append(pltpu.async_copy(
                        tab_sp.at[idx_v.at[r, pl.ds(s * _LANES, _LANES)]],
                        val_v.at[r, pl.ds(s * _LANES, _LANES)],
                        sem,
                    ))
            for d in descs:
                d.wait()
            pltpu.sync_copy(
                val_v, out_hbm.at[t, :, pl.ds(col0, block_cols)]
            )
            return carry

        lax.fori_loop(0, blocks_per_w, body, 0)

    return k


def kernel(v, idx):
    b, s = idx.shape
    out_t = _gather_call(s, b, v.shape[0], 512)(v, idx.T.astype(jnp.int32))
    return out_t.T
